# Initial kernel scaffold; baseline (speedup 1.0000x reference)
import jax
import jax.numpy as jnp
from jax.experimental import pallas as pl


def kernel(h, pos, segment_ids, pos_table):
    raise NotImplementedError("write your pallas kernel here")


# speedup vs baseline: 4.0336x; 4.0336x over previous
import functools

import jax
import jax.numpy as jnp
from jax import lax
from jax.experimental import pallas as pl
from jax.experimental.pallas import tpu as pltpu
from jax.experimental.pallas import tpu_sc as plsc

NC = 2
NS = 16
NW = NC * NS
L = 16

FB = 16
WPAD = 2 * L
BLK = 128


@functools.lru_cache(maxsize=None)
def _make_kernels(N, D, S):
    assert D % L == 0 and N % 8 == 0
    ND = D // L
    UNITS = N // 8
    CB = (UNITS // NW) * 8
    EXTRA = UNITS % NW
    CMAX = CB + (8 if EXTRA else 0)
    NFULL = CB // BLK
    TUNITS = (CMAX - NFULL * BLK) // 8
    assert NFULL >= 1 and CMAX - NFULL * BLK < BLK
    SLEN = ((CMAX + L + 15) // 16) * 16
    NPAD = (NW - 1) * CB + 8 * min(NW - 1, EXTRA) + SLEN
    TRASH = S
    SROWS = S + FB
    STRIPE = S // NS

    mesh = plsc.VectorSubcoreMesh(
        core_axis_name="c", subcore_axis_name="s", num_cores=NC, num_subcores=NS
    )

    def k1_body(seg_hbm, pos_hbm, h_hbm, w_hbm, num_out, den_out,
                num_acc, den_acc, h_buf, seg_v, pos_v, w_v,
                fnum, fden, zbuf, zbufd, sem0, sem1):
        cid = lax.axis_index("c")
        sid = lax.axis_index("s")
        wid = cid * NS + sid

        zero = jnp.zeros((L,), jnp.float32)
        lane = lax.iota(jnp.int32, L)
        trash_vec = jnp.full((L,), TRASH, jnp.int32)
        one_i = jnp.full((L,), 1, jnp.int32)
        onehot0 = (one_i - jnp.minimum(lane, one_i)).astype(jnp.float32)

        def zrow(r, _):
            zr, zdr = zbuf.at[r], zbufd.at[r]
            for k in range(ND):
                zr[pl.ds(k * L, L)] = zero
                zdr[pl.ds(k * L, L)] = zero
            return 0
        lax.fori_loop(0, STRIPE, zrow, 0)
        pltpu.sync_copy(zbuf, num_acc.at[pl.ds(sid * STRIPE, STRIPE)])
        pltpu.sync_copy(zbufd, den_acc.at[pl.ds(sid * STRIPE, STRIPE)])
        plsc.subcore_barrier()

        start = pl.multiple_of(wid * CB + 8 * jnp.minimum(wid, EXTRA), 8)
        cnt = CB + 8 * (wid < EXTRA).astype(jnp.int32)

        pltpu.sync_copy(seg_hbm.at[pl.ds(start, SLEN)], seg_v)
        pltpu.sync_copy(pos_hbm.at[pl.ds(start, SLEN)], pos_v)
        pltpu.sync_copy(w_hbm, w_v)

        sems = (sem0, sem1)
        def blk_start(b):
            return pl.multiple_of((start + b * BLK) * D, 8)

        pltpu.async_copy(h_hbm.at[pl.ds(blk_start(0), BLK * D)], h_buf.at[0],
                         sems[0])

        def flush(accs, dens, cur, cnt16):
            fr = fnum.at[cnt16]
            for k in range(ND):
                fr[pl.ds(k * L, L)] = accs[k]
            fd = fden.at[cnt16]
            fd[pl.ds(0, L)] = dens * onehot0

        def send_batch(idxv):
            pltpu.sync_copy(fnum, num_acc.at[idxv], add=True)
            pltpu.sync_copy(fden, den_acc.at[idxv], add=True)

        def make_row(hb, off, cutoff=None):
            def row(i, carry):
                dens, cur, cnt16, idxv, *accs = carry
                s = seg_v[pl.ds(i, L)][0]
                p = pos_v[pl.ds(i, L)][0]
                w = w_v[pl.ds(p, L)][0]
                if cutoff is not None:
                    valid = i < cutoff
                    s = jnp.where(valid, s, cur)
                    w = jnp.where(valid, w, 0.0)
                hbase = (i - off) * D
                newrun = s != cur

                @pl.when(newrun)
                def _():
                    flush(accs, dens, cur, cnt16)

                nr_i = newrun.astype(jnp.int32)
                m = (one_i - jnp.minimum(jnp.abs(lane - cnt16), one_i)) * nr_i
                idxv2 = idxv * (one_i - m) + cur * m
                cnt2 = jnp.where(newrun, cnt16 + 1, cnt16)
                full = cnt2 == FB

                @pl.when(full)
                def _():
                    send_batch(idxv2)

                cnt3 = jnp.where(full, 0, cnt2)
                f_i = full.astype(jnp.int32)
                idxv3 = idxv2 * (1 - f_i) + trash_vec * f_i
                naccs = [
                    jnp.where(newrun, 0.0, accs[k])
                    + w * hb[pl.ds(hbase + k * L, L)]
                    for k in range(ND)
                ]
                ndens = jnp.where(newrun, 0.0, dens) + w
                return (ndens, s, cnt3, idxv3, *naccs)
            return row

        def make_row_masked(hb, off, cutoff):
            return make_row(hb, off, cutoff)

        accs = [zero] * ND
        dens = jnp.float32(0.0)
        cur = seg_v[pl.ds(0, L)][0]
        carry = (dens, cur, jnp.int32(0), trash_vec, *accs)

        for b in range(NFULL):
            par = b % 2
            hb = h_buf.at[par]
            pltpu.make_async_copy(
                h_hbm.at[pl.ds(blk_start(b), BLK * D)], h_buf.at[par],
                sems[par]).wait()
            if b + 1 < NFULL:
                npar = (b + 1) % 2
                pltpu.async_copy(h_hbm.at[pl.ds(blk_start(b + 1), BLK * D)],
                                 h_buf.at[npar], sems[npar])
            carry = lax.fori_loop(b * BLK, (b + 1) * BLK,
                                  make_row(hb, b * BLK), carry)

        tpar = NFULL % 2
        tb = h_buf.at[tpar]
        for u in range(TUNITS):
            trow = NFULL * BLK + u * 8

            @pl.when(trow < cnt)
            def _(u=u, trow=trow):
                pltpu.sync_copy(
                    h_hbm.at[pl.ds(pl.multiple_of((start + trow) * D, 8),
                                   8 * D)],
                    tb.at[pl.ds(u * 8 * D, 8 * D)])
        carry = lax.fori_loop(NFULL * BLK, CMAX,
                              make_row_masked(tb, NFULL * BLK, cnt), carry)

        dens, cur, cnt16, idxv, *accs = carry
        flush(accs, dens, cur, cnt16)
        mf = one_i - jnp.minimum(jnp.abs(lane - cnt16), one_i)
        idxv = idxv * (one_i - mf) + cur * mf
        send_batch(idxv)

        plsc.subcore_barrier()
        pltpu.sync_copy(num_acc.at[pl.ds(sid * STRIPE, STRIPE)],
                        num_out.at[cid, pl.ds(sid * STRIPE, STRIPE)])
        pltpu.sync_copy(den_acc.at[pl.ds(sid * STRIPE, STRIPE)],
                        den_out.at[cid, pl.ds(sid * STRIPE, STRIPE)])

    k1 = pl.kernel(
        k1_body,
        out_type=(
            jax.ShapeDtypeStruct((NC, S, D), jnp.float32),
            jax.ShapeDtypeStruct((NC, S, D), jnp.float32),
        ),
        mesh=mesh,
        scratch_types=[
            pltpu.VMEM_SHARED((SROWS, D), jnp.float32),
            pltpu.VMEM_SHARED((SROWS, D), jnp.float32),
            pltpu.VMEM((2, BLK * D), jnp.float32),
            pltpu.VMEM((SLEN,), jnp.int32),
            pltpu.VMEM((SLEN,), jnp.int32),
            pltpu.VMEM((WPAD,), jnp.float32),
            pltpu.VMEM((FB, D), jnp.float32),
            pltpu.VMEM((FB, D), jnp.float32),
            pltpu.VMEM((STRIPE, D), jnp.float32),
            pltpu.VMEM((STRIPE, D), jnp.float32),
            pltpu.SemaphoreType.DMA,
            pltpu.SemaphoreType.DMA,
        ],
        name="wmean_segsum",
    )

    R2 = S // NW

    def k2_body(num_hbm, den_hbm, out_hbm, n0, n1, d0, d1, ov):
        cid = lax.axis_index("c")
        sid = lax.axis_index("s")
        wid = cid * NS + sid
        base = wid * R2
        pltpu.sync_copy(num_hbm.at[0, pl.ds(base, R2)], n0)
        pltpu.sync_copy(num_hbm.at[1, pl.ds(base, R2)], n1)
        pltpu.sync_copy(den_hbm.at[0, pl.ds(base, R2)], d0)
        pltpu.sync_copy(den_hbm.at[1, pl.ds(base, R2)], d1)

        def row(r, _):
            d = d0.at[r][pl.ds(0, L)] + d1.at[r][pl.ds(0, L)]
            inv = (1.0 / jnp.where(d > 0, d, 1.0))[0]
            n0r, n1r, ovr = n0.at[r], n1.at[r], ov.at[r]
            for k in range(ND):
                ovr[pl.ds(k * L, L)] = (
                    n0r[pl.ds(k * L, L)] + n1r[pl.ds(k * L, L)]
                ) * inv
            return 0
        lax.fori_loop(0, R2, row, 0)
        pltpu.sync_copy(ov, out_hbm.at[pl.ds(base, R2)])

    k2 = pl.kernel(
        k2_body,
        out_type=jax.ShapeDtypeStruct((S, D), jnp.float32),
        mesh=mesh,
        scratch_types=[
            pltpu.VMEM((R2, D), jnp.float32),
            pltpu.VMEM((R2, D), jnp.float32),
            pltpu.VMEM((R2, D), jnp.float32),
            pltpu.VMEM((R2, D), jnp.float32),
            pltpu.VMEM((R2, D), jnp.float32),
        ],
        name="wmean_combine",
    )

    return k1, k2, NPAD


def kernel(h, pos, segment_ids, pos_table):
    N, D = h.shape
    S = 1024
    k1, k2, npad = _make_kernels(N, D, S)

    seg32 = segment_ids.astype(jnp.int32)
    pos32 = pos.astype(jnp.int32)
    w = jax.nn.softplus(pos_table.astype(jnp.float32)).reshape(-1)
    wpad = jnp.pad(w, (0, WPAD - w.shape[0]))
    seg_p = jnp.pad(seg32, (0, npad - N))
    pos_p = jnp.pad(pos32, (0, npad - N))

    num, den = k1(seg_p, pos_p, h.reshape(-1), wpad)
    return k2(num, den)
